# baseline (speedup 1.0000x reference)
import functools
import jax
import jax.numpy as jnp
from jax import lax
from jax.experimental import pallas as pl
from jax.experimental.pallas import tpu as pltpu
from jax.experimental.pallas import tpu_sc as plsc

_N = 10000
_E = 320000
_HID = 128
_HEADS = 8
_HD = 16
_NC = 2
_NS = 16
_NW = _NC * _NS
_EPW = _E // _NW
_B = 48
_NB = 208
_BT = _EPW - _NB * _B
_NPAD = 10240
_RPS = _NPAD // _NS
_SROW = _NPAD // 8


def _proj_body(x_ref, win_ref, wq_ref, wk_ref, wv_ref, h_ref, q_ref, kv_ref):
    h = jnp.dot(x_ref[...], win_ref[...], preferred_element_type=jnp.float32)
    h_ref[...] = h
    q_ref[...] = jnp.dot(h, wq_ref[...], preferred_element_type=jnp.float32)
    k = jnp.dot(h, wk_ref[...], preferred_element_type=jnp.float32)
    v = jnp.dot(h, wv_ref[...], preferred_element_type=jnp.float32)
    kv_ref[...] = jnp.concatenate([k, v], axis=1)


def _edge_proj_body(a_ref, we_ref, e_ref):
    e_ref[...] = jnp.dot(a_ref[...], we_ref[...],
                         preferred_element_type=jnp.float32)


_GDN = lax.GatherDimensionNumbers(offset_dims=(), collapsed_slice_dims=(0,),
                                  start_index_map=(0,))


def _shuffle(x, idx):
    return lax.gather(x, idx[:, None], _GDN, slice_sizes=(1,),
                      mode=lax.GatherScatterMode.PROMISE_IN_BOUNDS)

def _sc_edge_body(q_hbm, kv_hbm, e_hbm, src_hbm, dst_hbm,
                  out_hbm, out2_hbm,
                  src_a, src_b, dst_a, dst_b, d8_a, d8_b,
                  src_t, dst_t, d8_t,
                  q_v, kv_v, em_v, sc_v,
                  acc_sh, sacc_sh, isem, gsem, ssem):
    c = lax.axis_index("c")
    s = lax.axis_index("s")
    ebase = (c * _NS + s) * _EPW

    zero16 = jnp.zeros((16,), jnp.float32)
    lane = lax.iota(jnp.int32, 16)

    idx_a = (src_a, dst_a, d8_a)
    idx_b = (src_b, dst_b, d8_b)

    def _zb(i, _):
        for buf in (em_v, sc_v):
            for j in range(_HID // 16):
                buf[i, pl.ds(j * 16, 16)] = zero16
        return 0
    lax.fori_loop(0, _B, _zb, 0)

    tgts = [acc_sh.at[pl.ds(s * _RPS + r * 40, 40), :]
            for r in range(_RPS // 40)]
    tgts += [sacc_sh.at[pl.ds(s * (_SROW // _NS) + r * 40, 40), :]
             for r in range(_SROW // _NS // 40)]
    for i0 in range(0, len(tgts), 6):
        cps = [pltpu.async_copy(em_v.at[pl.ds(0, 40), :], t, gsem)
               for t in tgts[i0:i0 + 6]]
        for cp in cps:
            cp.wait()
    plsc.subcore_barrier()

    def _idx_issue(b, idx, n):
        src_v, dst_v, d8_v = idx
        base = ebase + b * _B
        pltpu.async_copy(src_hbm.at[pl.ds(base, n)], src_v, isem)
        pltpu.async_copy(dst_hbm.at[pl.ds(base, n)], dst_v, isem)

    def _idx_wait(b, idx, n):
        src_v, dst_v, d8_v = idx
        base = ebase + b * _B
        pltpu.make_async_copy(src_hbm.at[pl.ds(base, n)], src_v, isem).wait()
        pltpu.make_async_copy(dst_hbm.at[pl.ds(base, n)], dst_v, isem).wait()
        for g in range(n // 16):
            d8_v[pl.ds(g * 16, 16)] = lax.shift_right_logical(
                dst_v[pl.ds(g * 16, 16)], 3)

    def _gather_issue(b, idx, n):
        src_v, dst_v, d8_v = idx
        base = ebase + b * _B
        pltpu.async_copy(q_hbm.at[dst_v], q_v.at[pl.ds(0, n), :], gsem)
        pltpu.async_copy(kv_hbm.at[src_v], kv_v.at[pl.ds(0, n), :], gsem)
        pltpu.async_copy(e_hbm.at[pl.ds(base, n), :],
                         em_v.at[pl.ds(0, n), :], gsem)

    def _gather_wait(b, idx, n):
        src_v, dst_v, d8_v = idx
        base = ebase + b * _B
        pltpu.make_async_copy(q_hbm.at[dst_v], q_v.at[pl.ds(0, n), :],
                              gsem).wait()
        pltpu.make_async_copy(kv_hbm.at[src_v], kv_v.at[pl.ds(0, n), :],
                              gsem).wait()
        pltpu.make_async_copy(e_hbm.at[pl.ds(base, n), :],
                              em_v.at[pl.ds(0, n), :], gsem).wait()

    def _compute(idx, n):
        src_v, dst_v, d8_v = idx

        @plsc.parallel_loop(0, n, 1, unroll=12)
        def _edge(i):
            g = (i // 16) * 16
            j = i - g
            dchunk = dst_v[pl.ds(g, 16)]
            d_bc = _shuffle(dchunk, lane * 0 + j)
            scores = zero16
            for h in range(_HEADS):
                qd = q_v[i, pl.ds(h * _HD, _HD)]
                eh = em_v[i, pl.ds(h * _HD, _HD)]
                ke = kv_v[i, pl.ds(h * _HD, _HD)] + eh
                prod = qd * ke
                for st in (1, 2, 4, 8):
                    prod = prod + _shuffle(prod, lane ^ st)
                sc = jnp.maximum(prod, 0.0) * 0.25
                ve = kv_v[i, pl.ds(_HID + h * _HD, _HD)] + eh
                em_v[i, pl.ds(h * _HD, _HD)] = sc * ve
                scores = jnp.where(lane == h, sc, scores)
            for j2 in range(_HID // 16):
                sc_v[i, pl.ds(j2 * 16, 16)] = zero16
            plsc.store_scatter(sc_v, [lane * 0 + i, (d_bc & 7) * 16 + lane],
                               scores)

        pltpu.async_copy(em_v.at[pl.ds(0, n), :], acc_sh.at[dst_v],
                         ssem, add=True)
        pltpu.async_copy(sc_v.at[pl.ds(0, n), :], sacc_sh.at[d8_v],
                         ssem, add=True)
        pltpu.make_async_copy(em_v.at[pl.ds(0, n), :], acc_sh.at[dst_v],
                              ssem).wait()
        pltpu.make_async_copy(sc_v.at[pl.ds(0, n), :], sacc_sh.at[d8_v],
                              ssem).wait()

    def _body(b, cur, nxt, nxt_n):
        if nxt is not None:
            _idx_issue(b + 1, nxt, nxt_n)
        _gather_wait(b, cur, _B)
        _compute(cur, _B)
        if nxt is not None:
            _idx_wait(b + 1, nxt, nxt_n)
            _gather_issue(b + 1, nxt, nxt_n)

    _idx_issue(0, idx_a, _B)
    _idx_wait(0, idx_a, _B)
    _gather_issue(0, idx_a, _B)

    def _pair(k, _):
        b0 = k * 2
        _body(b0, idx_a, idx_b, _B)
        _body(b0 + 1, idx_b, idx_a, _B)
        return 0
    lax.fori_loop(0, _NB // 2 - 1, _pair, 0)
    _body(_NB - 2, idx_a, idx_b, _B)
    _body(_NB - 1, idx_b, (src_t, dst_t, d8_t), _BT)
    _gather_wait(_NB, (src_t, dst_t, d8_t), _BT)
    _compute((src_t, dst_t, d8_t), _BT)

    plsc.subcore_barrier()
    pltpu.sync_copy(acc_sh.at[pl.ds(s * _RPS, _RPS), :],
                    out_hbm.at[c, pl.ds(s * _RPS, _RPS), :])
    pltpu.sync_copy(sacc_sh.at[pl.ds(s * (_SROW // _NS), _SROW // _NS), :],
                    out2_hbm.at[c, pl.ds(s * (_SROW // _NS), _SROW // _NS), :])


def _final_body(p_ref, p2_ref, h_ref, wout_ref, bout_ref, gnw_ref, gnb_ref,
                gna_ref, wcls_ref, bcls_ref, out_ref):
    unnorm = p_ref[0, :_N, :] + p_ref[1, :_N, :]
    rs = p2_ref[0, :_N, :] + p2_ref[1, :_N, :]
    col = lax.broadcasted_iota(jnp.int32, (16, _HID), 1)
    row = lax.broadcasted_iota(jnp.int32, (16, _HID), 0)
    sel = jnp.where((col // _HD) == row, 1.0, 0.0).astype(jnp.float32)
    denom = jnp.dot(rs, sel, preferred_element_type=jnp.float32) + 1e-6
    agg = unnorm / denom
    out = jnp.dot(agg, wout_ref[...], preferred_element_type=jnp.float32)
    out = out + bout_ref[...] + h_ref[...]
    mean = jnp.mean(out, axis=0, keepdims=True)
    shifted = out - gna_ref[...] * mean
    var = jnp.mean(shifted * shifted, axis=0, keepdims=True)
    out = gnw_ref[...] * shifted / jnp.sqrt(var + 1e-5) + gnb_ref[...]
    out = jnp.maximum(out, 0.0)
    out_ref[...] = jnp.dot(out, wcls_ref[...],
                           preferred_element_type=jnp.float32) + bcls_ref[...]


def kernel(x, edge_index, edge_attr, W_in, WQ, WK, WV, WE, W_out, b_out,
           gn_weight, gn_bias, gn_alpha, W_cls, b_cls):
    f32 = jnp.float32

    h, q, kv = pl.pallas_call(
        _proj_body,
        out_shape=[
            jax.ShapeDtypeStruct((_N, _HID), f32),
            jax.ShapeDtypeStruct((_N, _HID), f32),
            jax.ShapeDtypeStruct((_N, 2 * _HID), f32),
        ],
    )(x, W_in, WQ, WK, WV)

    e = pl.pallas_call(
        _edge_proj_body,
        grid=(16,),
        in_specs=[
            pl.BlockSpec((_E // 16, 16), lambda i: (i, 0)),
            pl.BlockSpec((16, _HID), lambda i: (0, 0)),
        ],
        out_specs=pl.BlockSpec((_E // 16, _HID), lambda i: (i, 0)),
        out_shape=jax.ShapeDtypeStruct((_E, _HID), f32),
    )(edge_attr, WE)

    sc_edge = functools.partial(
        pl.kernel,
        mesh=plsc.VectorSubcoreMesh(core_axis_name="c", subcore_axis_name="s"),
        out_type=[
            jax.ShapeDtypeStruct((_NC, _NPAD, _HID), f32),
            jax.ShapeDtypeStruct((_NC, _SROW, _HID), f32),
        ],
        scratch_types=(
            [pltpu.VMEM((_B,), jnp.int32)] * 6
            + [pltpu.VMEM((_BT,), jnp.int32)] * 3
            + [
                pltpu.VMEM((_B, _HID), f32),
                pltpu.VMEM((_B, 2 * _HID), f32),
                pltpu.VMEM((_B, _HID), f32),
                pltpu.VMEM((_B, _HID), f32),
                pltpu.VMEM_SHARED((_NPAD, _HID), f32),
                pltpu.VMEM_SHARED((_SROW, _HID), f32),
            ]
            + [pltpu.SemaphoreType.DMA] * 3
        ),
        compiler_params=pltpu.CompilerParams(needs_layout_passes=False),
    )(_sc_edge_body)
    partials, partials2 = sc_edge(q, kv, e, edge_index[0], edge_index[1])

    logits_pad = pl.pallas_call(
        _final_body,
        out_shape=jax.ShapeDtypeStruct((_N, _HID), f32),
    )(partials, partials2.reshape(_NC, _NPAD, 16), h, W_out,
      b_out.reshape(1, _HID),
      gn_weight.reshape(1, _HID), gn_bias.reshape(1, _HID),
      gn_alpha.reshape(1, _HID),
      jnp.pad(W_cls, ((0, 0), (0, _HID - 2))),
      jnp.pad(b_cls, (0, _HID - 2)).reshape(1, _HID))

    return logits_pad[:, :2]


# speedup vs baseline: 1.2936x; 1.2936x over previous
import functools
import jax
import jax.numpy as jnp
from jax import lax
from jax.experimental import pallas as pl
from jax.experimental.pallas import tpu as pltpu
from jax.experimental.pallas import tpu_sc as plsc

_N = 10000
_E = 320000
_HID = 128
_HEADS = 8
_HD = 16
_NC = 2
_NS = 16
_NW = _NC * _NS
_EPW = _E // _NW
_B = 48
_NB = 208
_BT = _EPW - _NB * _B
_NPAD = 10240
_RPS = _NPAD // _NS
_SROW = _NPAD // 8


def _proj_body(x_ref, win_ref, wq_ref, wk_ref, wv_ref, h_ref, q_ref, kv_ref):
    h = jnp.dot(x_ref[...], win_ref[...], preferred_element_type=jnp.float32)
    h_ref[...] = h
    q_ref[...] = jnp.dot(h, wq_ref[...], preferred_element_type=jnp.float32)
    k = jnp.dot(h, wk_ref[...], preferred_element_type=jnp.float32)
    v = jnp.dot(h, wv_ref[...], preferred_element_type=jnp.float32)
    kv_ref[...] = jnp.concatenate([k, v], axis=1)


def _edge_proj_body(a_ref, we_ref, e_ref):
    e_ref[...] = jnp.dot(a_ref[...], we_ref[...],
                         preferred_element_type=jnp.float32)


_GDN = lax.GatherDimensionNumbers(offset_dims=(), collapsed_slice_dims=(0,),
                                  start_index_map=(0,))


def _shuffle(x, idx):
    return lax.gather(x, idx[:, None], _GDN, slice_sizes=(1,),
                      mode=lax.GatherScatterMode.PROMISE_IN_BOUNDS)

def _sc_edge_body(q_hbm, kv_hbm, e_hbm, src_hbm, dst_hbm,
                  out_hbm, out2_hbm,
                  src_a, src_b, dst_a, dst_b, d8_a, d8_b,
                  src_t, dst_t, d8_t,
                  q_v, kv_v, em_a, em_b, sc_v,
                  acc_sh, sacc_sh, isem, gsem, ssem):
    c = lax.axis_index("c")
    s = lax.axis_index("s")
    ebase = (c * _NS + s) * _EPW

    zero16 = jnp.zeros((16,), jnp.float32)
    lane = lax.iota(jnp.int32, 16)

    idx_a = (src_a, dst_a, d8_a)
    idx_b = (src_b, dst_b, d8_b)

    def _zb(i, _):
        for buf in (em_a, em_b, sc_v):
            for j in range(_HID // 16):
                buf[i, pl.ds(j * 16, 16)] = zero16
        return 0
    lax.fori_loop(0, _B, _zb, 0)

    tgts = [acc_sh.at[pl.ds(s * _RPS + r * 40, 40), :]
            for r in range(_RPS // 40)]
    tgts += [sacc_sh.at[pl.ds(s * (_SROW // _NS) + r * 40, 40), :]
             for r in range(_SROW // _NS // 40)]
    for i0 in range(0, len(tgts), 6):
        cps = [pltpu.async_copy(em_a.at[pl.ds(0, 40), :], t, gsem)
               for t in tgts[i0:i0 + 6]]
        for cp in cps:
            cp.wait()
    plsc.subcore_barrier()

    def _idx_issue(b, idx, n):
        src_v, dst_v, d8_v = idx
        base = ebase + b * _B
        pltpu.async_copy(src_hbm.at[pl.ds(base, n)], src_v, isem)
        pltpu.async_copy(dst_hbm.at[pl.ds(base, n)], dst_v, isem)

    def _idx_wait(b, idx, n):
        src_v, dst_v, d8_v = idx
        base = ebase + b * _B
        pltpu.make_async_copy(src_hbm.at[pl.ds(base, n)], src_v, isem).wait()
        pltpu.make_async_copy(dst_hbm.at[pl.ds(base, n)], dst_v, isem).wait()
        for g in range(n // 16):
            d8_v[pl.ds(g * 16, 16)] = lax.shift_right_logical(
                dst_v[pl.ds(g * 16, 16)], 3)

    def _gather_issue(b, idx, em_v, n):
        src_v, dst_v, d8_v = idx
        base = ebase + b * _B
        pltpu.async_copy(q_hbm.at[dst_v], q_v.at[pl.ds(0, n), :], gsem)
        pltpu.async_copy(kv_hbm.at[src_v], kv_v.at[pl.ds(0, n), :], gsem)
        pltpu.async_copy(e_hbm.at[pl.ds(base, n), :],
                         em_v.at[pl.ds(0, n), :], gsem)

    def _gather_wait(b, idx, em_v, n):
        src_v, dst_v, d8_v = idx
        base = ebase + b * _B
        pltpu.make_async_copy(q_hbm.at[dst_v], q_v.at[pl.ds(0, n), :],
                              gsem).wait()
        pltpu.make_async_copy(kv_hbm.at[src_v], kv_v.at[pl.ds(0, n), :],
                              gsem).wait()
        pltpu.make_async_copy(e_hbm.at[pl.ds(base, n), :],
                              em_v.at[pl.ds(0, n), :], gsem).wait()

    def _scatter_wait(idx, em_v, n):
        src_v, dst_v, d8_v = idx
        pltpu.make_async_copy(em_v.at[pl.ds(0, n), :], acc_sh.at[dst_v],
                              ssem).wait()
        pltpu.make_async_copy(sc_v.at[pl.ds(0, n), :], sacc_sh.at[d8_v],
                              ssem).wait()

    def _compute(idx, em_v, n):
        src_v, dst_v, d8_v = idx

        @plsc.parallel_loop(0, n, 1, unroll=8)
        def _edge(i):
            g = (i // 16) * 16
            j = i - g
            dchunk = dst_v[pl.ds(g, 16)]
            d_bc = _shuffle(dchunk, lane * 0 + j)
            scores = zero16
            for h in range(_HEADS):
                qd = q_v[i, pl.ds(h * _HD, _HD)]
                eh = em_v[i, pl.ds(h * _HD, _HD)]
                ke = kv_v[i, pl.ds(h * _HD, _HD)] + eh
                prod = qd * ke
                for st in (1, 2, 4, 8):
                    prod = prod + _shuffle(prod, lane ^ st)
                sc = jnp.maximum(prod, 0.0) * 0.25
                ve = kv_v[i, pl.ds(_HID + h * _HD, _HD)] + eh
                em_v[i, pl.ds(h * _HD, _HD)] = sc * ve
                scores = jnp.where(lane == h, sc, scores)
            for j2 in range(_HID // 16):
                sc_v[i, pl.ds(j2 * 16, 16)] = zero16
            plsc.store_scatter(sc_v, [lane * 0 + i, (d_bc & 7) * 16 + lane],
                               scores)

        pltpu.async_copy(em_v.at[pl.ds(0, n), :], acc_sh.at[dst_v],
                         ssem, add=True)
        pltpu.async_copy(sc_v.at[pl.ds(0, n), :], sacc_sh.at[d8_v],
                         ssem, add=True)

    def _body(b, cur, nxt, em_cur, em_other, prev_idx, nxt_n, guard):
        if guard is None:
            _scatter_wait(prev_idx, em_other, _B)
        else:
            @pl.when(guard)
            def _():
                _scatter_wait(prev_idx, em_other, _B)
        _idx_issue(b + 1, nxt, nxt_n)
        _gather_wait(b, cur, em_cur, _B)
        _compute(cur, em_cur, _B)
        _idx_wait(b + 1, nxt, nxt_n)
        _gather_issue(b + 1, nxt, em_other, nxt_n)

    _idx_issue(0, idx_a, _B)
    _idx_wait(0, idx_a, _B)
    _gather_issue(0, idx_a, em_a, _B)

    def _pair(k, _):
        b0 = k * 2
        _body(b0, idx_a, idx_b, em_a, em_b, idx_b, _B, k >= 1)
        _body(b0 + 1, idx_b, idx_a, em_b, em_a, idx_a, _B, None)
        return 0
    lax.fori_loop(0, _NB // 2 - 1, _pair, 0)
    tidx = (src_t, dst_t, d8_t)
    _body(_NB - 2, idx_a, idx_b, em_a, em_b, idx_b, _B, None)
    _body(_NB - 1, idx_b, tidx, em_b, em_a, idx_a, _BT, None)
    _scatter_wait(idx_b, em_b, _B)
    _gather_wait(_NB, tidx, em_a, _BT)
    _compute(tidx, em_a, _BT)
    _scatter_wait(tidx, em_a, _BT)

    plsc.subcore_barrier()
    pltpu.sync_copy(acc_sh.at[pl.ds(s * _RPS, _RPS), :],
                    out_hbm.at[c, pl.ds(s * _RPS, _RPS), :])
    pltpu.sync_copy(sacc_sh.at[pl.ds(s * (_SROW // _NS), _SROW // _NS), :],
                    out2_hbm.at[c, pl.ds(s * (_SROW // _NS), _SROW // _NS), :])


def _final_body(p_ref, p2_ref, h_ref, wout_ref, bout_ref, gnw_ref, gnb_ref,
                gna_ref, wcls_ref, bcls_ref, out_ref):
    unnorm = p_ref[0, :_N, :] + p_ref[1, :_N, :]
    rs = p2_ref[0, :_N, :] + p2_ref[1, :_N, :]
    col = lax.broadcasted_iota(jnp.int32, (16, _HID), 1)
    row = lax.broadcasted_iota(jnp.int32, (16, _HID), 0)
    sel = jnp.where((col // _HD) == row, 1.0, 0.0).astype(jnp.float32)
    denom = jnp.dot(rs, sel, preferred_element_type=jnp.float32) + 1e-6
    agg = unnorm / denom
    out = jnp.dot(agg, wout_ref[...], preferred_element_type=jnp.float32)
    out = out + bout_ref[...] + h_ref[...]
    mean = jnp.mean(out, axis=0, keepdims=True)
    shifted = out - gna_ref[...] * mean
    var = jnp.mean(shifted * shifted, axis=0, keepdims=True)
    out = gnw_ref[...] * shifted / jnp.sqrt(var + 1e-5) + gnb_ref[...]
    out = jnp.maximum(out, 0.0)
    out_ref[...] = jnp.dot(out, wcls_ref[...],
                           preferred_element_type=jnp.float32) + bcls_ref[...]


def kernel(x, edge_index, edge_attr, W_in, WQ, WK, WV, WE, W_out, b_out,
           gn_weight, gn_bias, gn_alpha, W_cls, b_cls):
    f32 = jnp.float32

    h, q, kv = pl.pallas_call(
        _proj_body,
        out_shape=[
            jax.ShapeDtypeStruct((_N, _HID), f32),
            jax.ShapeDtypeStruct((_N, _HID), f32),
            jax.ShapeDtypeStruct((_N, 2 * _HID), f32),
        ],
    )(x, W_in, WQ, WK, WV)

    e = pl.pallas_call(
        _edge_proj_body,
        grid=(16,),
        in_specs=[
            pl.BlockSpec((_E // 16, 16), lambda i: (i, 0)),
            pl.BlockSpec((16, _HID), lambda i: (0, 0)),
        ],
        out_specs=pl.BlockSpec((_E // 16, _HID), lambda i: (i, 0)),
        out_shape=jax.ShapeDtypeStruct((_E, _HID), f32),
    )(edge_attr, WE)

    sc_edge = functools.partial(
        pl.kernel,
        mesh=plsc.VectorSubcoreMesh(core_axis_name="c", subcore_axis_name="s"),
        out_type=[
            jax.ShapeDtypeStruct((_NC, _NPAD, _HID), f32),
            jax.ShapeDtypeStruct((_NC, _SROW, _HID), f32),
        ],
        scratch_types=(
            [pltpu.VMEM((_B,), jnp.int32)] * 6
            + [pltpu.VMEM((_BT,), jnp.int32)] * 3
            + [
                pltpu.VMEM((_B, _HID), f32),
                pltpu.VMEM((_B, 2 * _HID), f32),
                pltpu.VMEM((_B, _HID), f32),
                pltpu.VMEM((_B, _HID), f32),
                pltpu.VMEM((_B, _HID), f32),
                pltpu.VMEM_SHARED((_NPAD, _HID), f32),
                pltpu.VMEM_SHARED((_SROW, _HID), f32),
            ]
            + [pltpu.SemaphoreType.DMA] * 3
        ),
        compiler_params=pltpu.CompilerParams(needs_layout_passes=False),
    )(_sc_edge_body)
    partials, partials2 = sc_edge(q, kv, e, edge_index[0], edge_index[1])

    logits_pad = pl.pallas_call(
        _final_body,
        out_shape=jax.ShapeDtypeStruct((_N, _HID), f32),
    )(partials, partials2.reshape(_NC, _NPAD, 16), h, W_out,
      b_out.reshape(1, _HID),
      gn_weight.reshape(1, _HID), gn_bias.reshape(1, _HID),
      gn_alpha.reshape(1, _HID),
      jnp.pad(W_cls, ((0, 0), (0, _HID - 2))),
      jnp.pad(b_cls, (0, _HID - 2)).reshape(1, _HID))

    return logits_pad[:, :2]
